# baseline (speedup 1.0000x reference)
import jax
import jax.numpy as jnp
from jax.experimental import pallas as pl
from jax.experimental.pallas import tpu as pltpu
from jax.experimental.pallas import tpu_sc as plsc

_NUM_LEVELS = 4
_K = 1024
_D = 256
_B = 8192
_TILE = 256
_GRID = _B // _TILE
_GW = 128


def _level_body(cbn_ref, xn_ref, x_ref, cb_ref, idx_ref, counts_ref):
    xl = x_ref[...]
    cb = cb_ref[...]
    xc = jax.lax.dot_general(xl, cb, (((1,), (1,)), ((), ())),
                             preferred_element_type=jnp.float32)
    d2 = (xn_ref[...] - 2.0 * xc) + cbn_ref[0, :][None, :]
    d = jnp.sqrt(jnp.maximum(d2, 0.0))
    m = jnp.min(d, axis=1, keepdims=True)
    jidx = jax.lax.broadcasted_iota(jnp.int32, d.shape, 1)
    idx = jnp.min(jnp.where(d == m, jidx, _K), axis=1)
    oh = (jidx == idx[:, None]).astype(jnp.float32)
    idx_ref[0, :] = idx
    counts_ref[0, 0, :] = jnp.sum(oh, axis=0)


def _assemble_body(x0_ref, x1_ref, x2_ref, x3_ref,
                   q0_ref, q1_ref, q2_ref, q3_ref,
                   rs_ref, es_ref, zh_ref):
    zh = None
    for l, (xr, qr) in enumerate(((x0_ref, q0_ref), (x1_ref, q1_ref),
                                  (x2_ref, q2_ref), (x3_ref, q3_ref))):
        q = qr[...]
        rs_ref[:, l * _D:(l + 1) * _D] = xr[...]
        es_ref[:, l * _D:(l + 1) * _D] = q
        zh = q if zh is None else zh + q
    zh_ref[...] = zh


def _count_body(p0_ref, p1_ref, p2_ref, p3_ref, cnt_ref):
    total = jnp.int32(0)
    for p in (p0_ref, p1_ref, p2_ref, p3_ref):
        s = jnp.sum(p[...], axis=(0, 1))
        total += jnp.sum((s == 0.0).astype(jnp.int32))
    cnt_ref[0, 0] = total


def _level_call(xl, xn, cb, cbn):
    out_shapes = (
        jax.ShapeDtypeStruct((1, _B), jnp.int32),
        jax.ShapeDtypeStruct((_GRID, 1, _K), jnp.float32),
    )
    return pl.pallas_call(
        _level_body,
        grid=(_GRID,),
        in_specs=[
            pl.BlockSpec((1, _K), lambda i: (0, 0)),
            pl.BlockSpec((_TILE, 1), lambda i: (i, 0)),
            pl.BlockSpec((_TILE, _D), lambda i: (i, 0)),
            pl.BlockSpec((_K, _D), lambda i: (0, 0)),
        ],
        out_specs=[
            pl.BlockSpec((1, _TILE), lambda i: (0, i)),
            pl.BlockSpec((1, 1, _K), lambda i: (i, 0, 0)),
        ],
        out_shape=out_shapes,
        compiler_params=pltpu.CompilerParams(
            dimension_semantics=("parallel",)),
    )(cbn.reshape(1, _K), xn, xl, cb)


def _sc_gather(cb, idx2d):
    mesh = plsc.VectorSubcoreMesh(core_axis_name="c", subcore_axis_name="s")

    @pl.kernel(out_type=jax.ShapeDtypeStruct((_B, _D), jnp.float32),
               mesh=mesh)
    def gather_kernel(cb_hbm, i_hbm, o_hbm):
        def body(i_vmem, o_vmem):
            pltpu.sync_copy(cb_hbm.at[i_vmem.at[0]], o_vmem)

        pltpu.emit_pipeline(
            body,
            grid=(_B // _GW,),
            in_specs=[pl.BlockSpec((1, _GW), lambda i: (0, i))],
            out_specs=[pl.BlockSpec((_GW, _D), lambda i: (i, 0))],
            core_axis_name=("c", "s"),
            dimension_semantics=(pltpu.PARALLEL,),
        )(i_hbm, o_hbm)

    return gather_kernel(cb, idx2d)


def kernel(x, cb0, cb1, cb2, cb3):
    cbs = (cb0, cb1, cb2, cb3)
    cbns = [jnp.sum(cb * cb, axis=1) for cb in cbs]

    xl = x
    idxs, qs, partials, rs = [], [], [], []
    for l in range(_NUM_LEVELS):
        xn = jnp.sum(xl * xl, axis=1, keepdims=True)
        idx_l, counts_l = _level_call(xl, xn, cbs[l], cbns[l])
        q_l = _sc_gather(cbs[l], idx_l)
        partials.append(counts_l)
        idxs.append(idx_l[0])
        rs.append(xl)
        xl = xl - q_l
        qs.append(q_l)

    rs_flat, es_flat, zh = pl.pallas_call(
        _assemble_body,
        grid=(_GRID,),
        in_specs=[pl.BlockSpec((_TILE, _D), lambda i: (i, 0))] * 8,
        out_specs=[
            pl.BlockSpec((_TILE, _NUM_LEVELS * _D), lambda i: (i, 0)),
            pl.BlockSpec((_TILE, _NUM_LEVELS * _D), lambda i: (i, 0)),
            pl.BlockSpec((_TILE, _D), lambda i: (i, 0)),
        ],
        out_shape=(
            jax.ShapeDtypeStruct((_B, _NUM_LEVELS * _D), jnp.float32),
            jax.ShapeDtypeStruct((_B, _NUM_LEVELS * _D), jnp.float32),
            jax.ShapeDtypeStruct((_B, _D), jnp.float32),
        ),
        compiler_params=pltpu.CompilerParams(
            dimension_semantics=("parallel",)),
    )(*rs, *qs)

    cnt = pl.pallas_call(
        _count_body,
        in_specs=[pl.BlockSpec((_GRID, 1, _K), lambda: (0, 0, 0))] * 4,
        out_specs=pl.BlockSpec((1, 1), lambda: (0, 0),
                               memory_space=pltpu.SMEM),
        out_shape=jax.ShapeDtypeStruct((1, 1), jnp.int32),
    )(*partials)

    output = jnp.stack(idxs, axis=1).astype(jnp.int64)
    r_s = rs_flat.reshape(_B, _NUM_LEVELS, _D)
    e_s = es_flat.reshape(_B, _NUM_LEVELS, _D)
    count = cnt[0, 0]
    return output, r_s, e_s, zh, count


# speedup vs baseline: 1.1573x; 1.1573x over previous
import jax
import jax.numpy as jnp
from jax.experimental import pallas as pl
from jax.experimental.pallas import tpu as pltpu
from jax.experimental.pallas import tpu_sc as plsc

_NUM_LEVELS = 4
_K = 1024
_D = 256
_B = 8192
_TILE = 512
_GRID = _B // _TILE
_GW = 128


def _level_body(cbn_ref, xn_ref, x_ref, cb_ref, idx_ref, counts_ref):
    xl = x_ref[...]
    cb = cb_ref[...]
    xc = jax.lax.dot_general(xl, cb, (((1,), (1,)), ((), ())),
                             preferred_element_type=jnp.float32)
    d2 = (xn_ref[...] - 2.0 * xc) + cbn_ref[0, :][None, :]
    d = jnp.sqrt(jnp.maximum(d2, 0.0))
    m = jnp.min(d, axis=1, keepdims=True)
    jidx = jax.lax.broadcasted_iota(jnp.int32, d.shape, 1)
    idx = jnp.min(jnp.where(d == m, jidx, _K), axis=1)
    oh = (jidx == idx[:, None]).astype(jnp.float32)
    idx_ref[0, :] = idx
    counts_ref[0, 0, :] = jnp.sum(oh, axis=0)


def _epilogue_body(q0_ref, q1_ref, q2_ref, q3_ref,
                   p0_ref, p1_ref, p2_ref, p3_ref, zh_ref, cnt_ref):
    zh_ref[...] = ((q0_ref[...] + q1_ref[...]) + q2_ref[...]) + q3_ref[...]

    @pl.when(pl.program_id(0) == pl.num_programs(0) - 1)
    def _():
        total = jnp.int32(0)
        for p in (p0_ref, p1_ref, p2_ref, p3_ref):
            s = jnp.sum(p[...], axis=(0, 1))
            total += jnp.sum((s == 0.0).astype(jnp.int32))
        cnt_ref[0, 0] = total


def _level_call(xl, xn, cb, cbn):
    out_shapes = (
        jax.ShapeDtypeStruct((1, _B), jnp.int32),
        jax.ShapeDtypeStruct((_GRID, 1, _K), jnp.float32),
    )
    return pl.pallas_call(
        _level_body,
        grid=(_GRID,),
        in_specs=[
            pl.BlockSpec((1, _K), lambda i: (0, 0)),
            pl.BlockSpec((_TILE, 1), lambda i: (i, 0)),
            pl.BlockSpec((_TILE, _D), lambda i: (i, 0)),
            pl.BlockSpec((_K, _D), lambda i: (0, 0)),
        ],
        out_specs=[
            pl.BlockSpec((1, _TILE), lambda i: (0, i)),
            pl.BlockSpec((1, 1, _K), lambda i: (i, 0, 0)),
        ],
        out_shape=out_shapes,
        compiler_params=pltpu.CompilerParams(
            dimension_semantics=("parallel",)),
    )(cbn.reshape(1, _K), xn, xl, cb)


def _sc_gather(cb, idx2d):
    mesh = plsc.VectorSubcoreMesh(core_axis_name="c", subcore_axis_name="s")

    @pl.kernel(out_type=jax.ShapeDtypeStruct((_B, _D), jnp.float32),
               mesh=mesh)
    def gather_kernel(cb_hbm, i_hbm, o_hbm):
        def body(i_vmem, o_vmem):
            pltpu.sync_copy(cb_hbm.at[i_vmem.at[0]], o_vmem)

        pltpu.emit_pipeline(
            body,
            grid=(_B // _GW,),
            in_specs=[pl.BlockSpec((1, _GW), lambda i: (0, i))],
            out_specs=[pl.BlockSpec((_GW, _D), lambda i: (i, 0))],
            core_axis_name=("c", "s"),
            dimension_semantics=(pltpu.PARALLEL,),
        )(i_hbm, o_hbm)

    return gather_kernel(cb, idx2d)


def kernel(x, cb0, cb1, cb2, cb3):
    cbs = (cb0, cb1, cb2, cb3)
    cbns = [jnp.sum(cb * cb, axis=1) for cb in cbs]

    xl = x
    idxs, qs, partials, rs = [], [], [], []
    for l in range(_NUM_LEVELS):
        xn = jnp.sum(xl * xl, axis=1, keepdims=True)
        idx_l, counts_l = _level_call(xl, xn, cbs[l], cbns[l])
        q_l = _sc_gather(cbs[l], idx_l)
        partials.append(counts_l)
        idxs.append(idx_l[0])
        rs.append(xl)
        xl = xl - q_l
        qs.append(q_l)

    zh, cnt = pl.pallas_call(
        _epilogue_body,
        grid=(_GRID,),
        in_specs=[pl.BlockSpec((_TILE, _D), lambda i: (i, 0))] * 4
        + [pl.BlockSpec((_GRID, 1, _K), lambda i: (0, 0, 0))] * 4,
        out_specs=[
            pl.BlockSpec((_TILE, _D), lambda i: (i, 0)),
            pl.BlockSpec((1, 1), lambda i: (0, 0),
                         memory_space=pltpu.SMEM),
        ],
        out_shape=(
            jax.ShapeDtypeStruct((_B, _D), jnp.float32),
            jax.ShapeDtypeStruct((1, 1), jnp.int32),
        ),
    )(*qs, *partials)

    output = jnp.stack(idxs, axis=1).astype(jnp.int64)
    r_s = jnp.stack(rs, axis=1)
    e_s = jnp.stack(qs, axis=1)
    count = cnt[0, 0]
    return output, r_s, e_s, zh, count


# speedup vs baseline: 1.1605x; 1.0027x over previous
import jax
import jax.numpy as jnp
from jax.experimental import pallas as pl
from jax.experimental.pallas import tpu as pltpu
from jax.experimental.pallas import tpu_sc as plsc

_NUM_LEVELS = 4
_K = 1024
_D = 256
_B = 8192
_TILE = 1024
_GRID = _B // _TILE
_GW = 128


def _level_body(cbn_ref, xn_ref, x_ref, cb_ref, idx_ref, counts_ref):
    xl = x_ref[...]
    cb = cb_ref[...]
    xc = jax.lax.dot_general(xl, cb, (((1,), (1,)), ((), ())),
                             preferred_element_type=jnp.float32)
    d2 = (xn_ref[...] - 2.0 * xc) + cbn_ref[0, :][None, :]
    d = jnp.sqrt(jnp.maximum(d2, 0.0))
    m = jnp.min(d, axis=1, keepdims=True)
    jidx = jax.lax.broadcasted_iota(jnp.int32, d.shape, 1)
    idx = jnp.min(jnp.where(d == m, jidx, _K), axis=1)
    oh = (jidx == idx[:, None]).astype(jnp.float32)
    idx_ref[0, :] = idx
    counts_ref[0, 0, :] = jnp.sum(oh, axis=0)


def _epilogue_body(q0_ref, q1_ref, q2_ref, q3_ref,
                   p0_ref, p1_ref, p2_ref, p3_ref, zh_ref, cnt_ref):
    zh_ref[...] = ((q0_ref[...] + q1_ref[...]) + q2_ref[...]) + q3_ref[...]

    @pl.when(pl.program_id(0) == pl.num_programs(0) - 1)
    def _():
        total = jnp.int32(0)
        for p in (p0_ref, p1_ref, p2_ref, p3_ref):
            s = jnp.sum(p[...], axis=(0, 1))
            total += jnp.sum((s == 0.0).astype(jnp.int32))
        cnt_ref[0, 0] = total


def _level_call(xl, xn, cb, cbn):
    out_shapes = (
        jax.ShapeDtypeStruct((1, _B), jnp.int32),
        jax.ShapeDtypeStruct((_GRID, 1, _K), jnp.float32),
    )
    return pl.pallas_call(
        _level_body,
        grid=(_GRID,),
        in_specs=[
            pl.BlockSpec((1, _K), lambda i: (0, 0)),
            pl.BlockSpec((_TILE, 1), lambda i: (i, 0)),
            pl.BlockSpec((_TILE, _D), lambda i: (i, 0)),
            pl.BlockSpec((_K, _D), lambda i: (0, 0)),
        ],
        out_specs=[
            pl.BlockSpec((1, _TILE), lambda i: (0, i)),
            pl.BlockSpec((1, 1, _K), lambda i: (i, 0, 0)),
        ],
        out_shape=out_shapes,
        compiler_params=pltpu.CompilerParams(
            dimension_semantics=("parallel",)),
    )(cbn.reshape(1, _K), xn, xl, cb)


def _sc_gather(cb, idx2d):
    mesh = plsc.VectorSubcoreMesh(core_axis_name="c", subcore_axis_name="s")

    @pl.kernel(out_type=jax.ShapeDtypeStruct((_B, _D), jnp.float32),
               mesh=mesh)
    def gather_kernel(cb_hbm, i_hbm, o_hbm):
        def body(i_vmem, o_vmem):
            pltpu.sync_copy(cb_hbm.at[i_vmem.at[0]], o_vmem)

        pltpu.emit_pipeline(
            body,
            grid=(_B // _GW,),
            in_specs=[pl.BlockSpec((1, _GW), lambda i: (0, i))],
            out_specs=[pl.BlockSpec((_GW, _D), lambda i: (i, 0))],
            core_axis_name=("c", "s"),
            dimension_semantics=(pltpu.PARALLEL,),
        )(i_hbm, o_hbm)

    return gather_kernel(cb, idx2d)


def kernel(x, cb0, cb1, cb2, cb3):
    cbs = (cb0, cb1, cb2, cb3)
    cbns = [jnp.sum(cb * cb, axis=1) for cb in cbs]

    xl = x
    idxs, qs, partials, rs = [], [], [], []
    for l in range(_NUM_LEVELS):
        xn = jnp.sum(xl * xl, axis=1, keepdims=True)
        idx_l, counts_l = _level_call(xl, xn, cbs[l], cbns[l])
        q_l = _sc_gather(cbs[l], idx_l)
        partials.append(counts_l)
        idxs.append(idx_l[0])
        rs.append(xl)
        xl = xl - q_l
        qs.append(q_l)

    zh, cnt = pl.pallas_call(
        _epilogue_body,
        grid=(_GRID,),
        in_specs=[pl.BlockSpec((_TILE, _D), lambda i: (i, 0))] * 4
        + [pl.BlockSpec((_GRID, 1, _K), lambda i: (0, 0, 0))] * 4,
        out_specs=[
            pl.BlockSpec((_TILE, _D), lambda i: (i, 0)),
            pl.BlockSpec((1, 1), lambda i: (0, 0),
                         memory_space=pltpu.SMEM),
        ],
        out_shape=(
            jax.ShapeDtypeStruct((_B, _D), jnp.float32),
            jax.ShapeDtypeStruct((1, 1), jnp.int32),
        ),
    )(*qs, *partials)

    output = jnp.stack(idxs, axis=1).astype(jnp.int64)
    r_s = jnp.stack(rs, axis=1)
    e_s = jnp.stack(qs, axis=1)
    count = cnt[0, 0]
    return output, r_s, e_s, zh, count
